# baseline (speedup 1.0000x reference)
import jax
import jax.numpy as jnp
from jax import lax
from jax.experimental import pallas as pl
from jax.experimental.pallas import tpu as pltpu
from jax.experimental.pallas import tpu_sc as plsc

EMB = 128
CH = 16
NCH = EMB // CH
NV = 50000
NVP = 50176
RPT = NVP // 16
K = 512
KJ = K // 128
M0 = 49
M1 = 25
EP0 = 16 * K * M0
EP1 = 16 * K * M1
EP = EP0 + EP1
NC0 = EP0 // K
MD = EP // 16 // K

_MESH = dict(core_axis_name="c", subcore_axis_name="s",
             num_cores=2, num_subcores=16)
_CPARAMS = pltpu.CompilerParams(
    needs_layout_passes=False, use_tc_tiling_on_sc=False)


def _sc_scales(pidx, wts):
    def body(pidx_ref, wts_ref, sc2_ref,
             pb, wbuf, rud, rid, scb, dtmp, smg, ud_s, id_s):
        core = lax.axis_index("c")
        sub = lax.axis_index("s")
        tb = sub * RPT
        z16 = jnp.zeros((16,), jnp.float32)

        @pl.loop(0, RPT // 16)
        def _(g):
            dtmp[pl.ds(g * 16, 16)] = z16

        pltpu.sync_copy(dtmp, ud_s.at[pl.ds(tb, RPT)])
        pltpu.sync_copy(dtmp, id_s.at[pl.ds(tb, RPT)])
        plsc.subcore_barrier()

        @pl.loop(0, MD)
        def _(c):
            ch = sub * MD + c
            d1 = pltpu.async_copy(pidx_ref.at[ch], pb, smg)
            d2 = pltpu.async_copy(wts_ref.at[pl.ds(ch * K, K)], wbuf, smg)
            d1.wait()
            d2.wait()
            d3 = pltpu.async_copy(wbuf, ud_s.at[pb.at[0]], smg, add=True)
            d4 = pltpu.async_copy(wbuf, id_s.at[pb.at[1]], smg, add=True)
            d3.wait()
            d4.wait()

        plsc.subcore_barrier()

        def rsqrt16(v):
            i = lax.bitcast_convert_type(v, jnp.int32)
            i = jnp.int32(0x5F3759DF) - lax.shift_right_logical(i, 1)
            y = lax.bitcast_convert_type(i, jnp.float32)
            for _ in range(3):
                y = y * (1.5 - 0.5 * v * y * y)
            return y

        for dref in (ud_s, id_s):
            pltpu.sync_copy(dref.at[pl.ds(tb, RPT)], dtmp)

            @pl.loop(0, RPT // 16)
            def _(g):
                v = jnp.maximum(dtmp[pl.ds(g * 16, 16)], 1.0)
                dtmp[pl.ds(g * 16, 16)] = rsqrt16(v)

            pltpu.sync_copy(dtmp, dref.at[pl.ds(tb, RPT)])
        plsc.subcore_barrier()

        @pl.loop(0, MD)
        def _(c):
            ch = sub * MD + c
            d1 = pltpu.async_copy(pidx_ref.at[ch], pb, smg)
            d2 = pltpu.async_copy(wts_ref.at[pl.ds(ch * K, K)], wbuf, smg)
            d1.wait()
            d2.wait()
            d3 = pltpu.async_copy(ud_s.at[pb.at[0]], rud, smg)
            d4 = pltpu.async_copy(id_s.at[pb.at[1]], rid, smg)
            d3.wait()
            d4.wait()

            @pl.loop(0, K // 16)
            def _(g):
                sl = pl.ds(g * 16, 16)
                scb[sl] = wbuf[sl] * rud[sl] * rid[sl]

            pltpu.sync_copy(scb, sc2_ref.at[core, pl.ds(ch * K, K)])

    fn = pl.kernel(
        body,
        out_type=jax.ShapeDtypeStruct((2, EP), jnp.float32),
        mesh=plsc.VectorSubcoreMesh(**_MESH),
        compiler_params=_CPARAMS,
        scratch_types=[
            pltpu.VMEM((2, K), jnp.int32),
            pltpu.VMEM((K,), jnp.float32),
            pltpu.VMEM((K,), jnp.float32),
            pltpu.VMEM((K,), jnp.float32),
            pltpu.VMEM((K,), jnp.float32),
            pltpu.VMEM((RPT,), jnp.float32),
            pltpu.SemaphoreType.DMA,
            pltpu.VMEM_SHARED((NVP,), jnp.float32),
            pltpu.VMEM_SHARED((NVP,), jnp.float32),
        ],
    )
    return fn(pidx, wts)


def _sc_messages(pidx, sc2, rflat, ucat, icat):
    def body(pidx_ref, sc2_ref, rflat_ref, ucat_ref, icat_ref, msg_ref,
             pbA, pbB, rwA, rwB, scA, scB, rbuf,
             sIA, sIB, sRA, sRB, sOA, sOB, smg, acc):
        core = lax.axis_index("c")
        sub = lax.axis_index("s")
        tb = sub * RPT
        z16 = jnp.zeros((16,), jnp.float32)

        pltpu.sync_copy(rflat_ref, rbuf)

        def edge_loop(n, chbase, tbl, dsel, gsel, out_side, rv, poff):
            BS = ((pbA, scA, rwA, sIA, sRA, sOA),
                  (pbB, scB, rwB, sIB, sRB, sOB))

            def issue_idx(c, bs):
                pb, sc, rw, sI, sR, sO = bs
                ch = chbase + c
                pltpu.async_copy(pidx_ref.at[ch], pb, sI)
                pltpu.async_copy(
                    sc2_ref.at[out_side, pl.ds(ch * K, K)], sc, sI)

            def wait_idx(bs):
                pb, sc, rw, sI, sR, sO = bs
                pltpu.make_async_copy(pidx_ref.at[chbase], pb, sI).wait()
                pltpu.make_async_copy(
                    sc2_ref.at[out_side, pl.ds(chbase * K, K)], sc, sI).wait()

            def issue_gathers(bs):
                pb, sc, rw, sI, sR, sO = bs
                pltpu.async_copy(tbl.at[pb.at[gsel]], rw, sR)

            def wait_gathers(bs):
                pb, sc, rw, sI, sR, sO = bs
                pltpu.make_async_copy(tbl.at[pb.at[gsel]], rw, sR).wait()

            def issue_scatter(bs):
                pb, sc, rw, sI, sR, sO = bs
                pltpu.async_copy(rw, acc.at[pb.at[dsel]], sO, add=True)

            def wait_scatter(bs):
                pb, sc, rw, sI, sR, sO = bs
                pltpu.make_async_copy(rw, acc.at[pb.at[dsel]], sO).wait()

            def compute(bs):
                pb, sc, rw, sI, sR, sO = bs

                @pl.loop(0, K // 16)
                def _(g):
                    sv16 = sc[pl.ds(g * 16, 16)]
                    for l in range(16):
                        r = g * 16 + l
                        sv = lax.broadcast(sv16[l], (16,))
                        rw[r, pl.ds(0, 16)] = rw[r, pl.ds(0, 16)] * (sv * rv)

            def adjust_idx(bs):
                pb = bs[0]

                @pl.loop(0, K // 16)
                def _(g):
                    sl = pl.ds(g * 16, 16)
                    v = lax.shift_left(pb[gsel, sl], 3)
                    pb[gsel, sl] = v + poff

            def half(c, bs, bs_next, first, last):
                if not first:
                    wait_scatter(bs_next)
                if not last:
                    issue_idx(c + 1, bs_next)
                wait_gathers(bs)
                compute(bs)
                issue_scatter(bs)
                if not last:
                    wait_idx(bs_next)
                    adjust_idx(bs_next)
                    issue_gathers(bs_next)

            issue_idx(0, BS[0])
            wait_idx(BS[0])
            adjust_idx(BS[0])
            issue_gathers(BS[0])
            half(0, BS[0], BS[1], first=True, last=False)

            @pl.loop(0, (n - 2) // 2)
            def _(jj):
                c = 1 + 2 * jj
                half(c, BS[1], BS[0], first=False, last=False)
                half(c + 1, BS[0], BS[1], first=False, last=False)

            if n % 2 == 0:
                half(n - 1, BS[1], BS[0], first=False, last=True)
                wait_scatter(BS[1])
            else:
                half(n - 2, BS[1], BS[0], first=False, last=False)
                half(n - 1, BS[0], BS[1], first=False, last=True)
                wait_scatter(BS[0])

        def side(tbl, dsel, gsel, out_side):
            nfull = RPT // K
            rem = RPT - nfull * K

            @pl.loop(0, NCH)
            def _(p):
                poff = lax.broadcast(p, (16,)).astype(jnp.int32)

                @pl.loop(0, K)
                def _(r):
                    rwA[r, pl.ds(0, 16)] = z16

                ds_ = []
                for q in range(nfull):
                    ds_.append(pltpu.async_copy(
                        rwA, acc.at[pl.ds(tb + q * K, K)], smg))
                if rem:
                    ds_.append(pltpu.async_copy(
                        rwA.at[pl.ds(0, rem)],
                        acc.at[pl.ds(tb + nfull * K, rem)], smg))
                for d in ds_:
                    d.wait()
                plsc.subcore_barrier()

                rv_t0 = rbuf[pl.ds(p * CH, 16)]
                rv_t1 = rbuf[pl.ds(EMB + p * CH, 16)]
                edge_loop(M0, sub * M0, tbl, dsel, gsel, out_side, rv_t0,
                          poff)
                edge_loop(M1, NC0 + sub * M1, tbl, dsel, gsel, out_side,
                          rv_t1, poff)
                plsc.subcore_barrier()
                pltpu.sync_copy(acc.at[pl.ds(tb, RPT)],
                                msg_ref.at[out_side, p, pl.ds(tb, RPT)])

        @pl.when(core == 0)
        def _():
            side(icat_ref, 0, 1, 0)

        @pl.when(core == 1)
        def _():
            side(ucat_ref, 1, 0, 1)

    fn = pl.kernel(
        body,
        out_type=jax.ShapeDtypeStruct((2, NCH, NVP, CH), jnp.float32),
        mesh=plsc.VectorSubcoreMesh(**_MESH),
        compiler_params=_CPARAMS,
        scratch_types=[
            pltpu.VMEM((2, K), jnp.int32),
            pltpu.VMEM((2, K), jnp.int32),
            pltpu.VMEM((K, CH), jnp.float32),
            pltpu.VMEM((K, CH), jnp.float32),
            pltpu.VMEM((K,), jnp.float32),
            pltpu.VMEM((K,), jnp.float32),
            pltpu.VMEM((2 * EMB,), jnp.float32),
            pltpu.SemaphoreType.DMA,
            pltpu.SemaphoreType.DMA,
            pltpu.SemaphoreType.DMA,
            pltpu.SemaphoreType.DMA,
            pltpu.SemaphoreType.DMA,
            pltpu.SemaphoreType.DMA,
            pltpu.SemaphoreType.DMA,
            pltpu.VMEM_SHARED((NVP, CH), jnp.float32),
        ],
    )
    return fn(pidx, sc2, rflat, ucat, icat)


def _tc_matmul_body(x_ref, w_ref, ou_ref, oi_ref):
    for s, o_ref in ((0, ou_ref), (1, oi_ref)):
        x = x_ref[s]
        m = jnp.concatenate([x[p] for p in range(NCH)], axis=1)
        y = lax.dot_general(m, w_ref[s], (((1,), (1,)), ((), ())),
                            preferred_element_type=jnp.float32)
        o_ref[...] = jax.nn.sigmoid(y)


def _tc_rel_body(r_ref, w_ref, o_ref):
    o_ref[...] = lax.dot_general(r_ref[...], w_ref[...],
                                 (((1,), (1,)), ((), ())),
                                 preferred_element_type=jnp.float32)


def kernel(u_emb, i_emb, r_emb, edge_t0_index, edge_t0_weights,
           edge_t1_index, edge_t1_weights, W_u, W_i, W_rel):
    def pad_edges(idx, w, ept):
        e = w.shape[0]
        spread = (jnp.arange(ept - e, dtype=jnp.int32) * 97) % NV
        pu = jnp.concatenate([idx[0], spread])
        pi = jnp.concatenate([idx[1], spread])
        pw = jnp.concatenate([w, jnp.zeros((ept - e,), jnp.float32)])
        return pu, pi, pw

    pu0, pi0, pw0 = pad_edges(edge_t0_index, edge_t0_weights, EP0)
    pu1, pi1, pw1 = pad_edges(edge_t1_index, edge_t1_weights, EP1)
    uidx = jnp.concatenate([pu0, pu1])
    iidx = jnp.concatenate([pi0, pi1])
    pidx = (jnp.stack([uidx, iidx])
            .reshape(2, EP // K, K)
            .transpose(1, 0, 2))
    wts = jnp.concatenate([pw0, pw1])
    rflat = r_emb.reshape(2 * EMB)
    ucat = u_emb.reshape(NCH * NV, CH)
    icat = i_emb.reshape(NCH * NV, CH)

    sc2 = _sc_scales(pidx, wts)
    msg = _sc_messages(pidx, sc2, rflat, ucat, icat)

    B = 1024
    nblk = NVP // B
    w_st = jnp.stack([W_u, W_i])
    new_u, new_i = pl.pallas_call(
        _tc_matmul_body,
        grid=(nblk,),
        in_specs=[
            pl.BlockSpec((2, NCH, B, CH), lambda i: (0, 0, i, 0)),
            pl.BlockSpec((2, EMB, EMB), lambda i: (0, 0, 0)),
        ],
        out_specs=[
            pl.BlockSpec((B, EMB), lambda i: (i, 0)),
            pl.BlockSpec((B, EMB), lambda i: (i, 0)),
        ],
        out_shape=[
            jax.ShapeDtypeStruct((NV, EMB), jnp.float32),
            jax.ShapeDtypeStruct((NV, EMB), jnp.float32),
        ],
    )(msg, w_st)

    new_r = pl.pallas_call(
        _tc_rel_body,
        out_shape=jax.ShapeDtypeStruct((2, EMB), jnp.float32),
    )(r_emb, W_rel)

    return new_u, new_i, new_r


# speedup vs baseline: 1.0649x; 1.0649x over previous
import jax
import jax.numpy as jnp
from jax import lax
from jax.experimental import pallas as pl
from jax.experimental.pallas import tpu as pltpu
from jax.experimental.pallas import tpu_sc as plsc

EMB = 128
CH = 16
NCH = EMB // CH
NV = 50000
NVP = 50176
RPT = NVP // 16
K = 512
KJ = K // 128
M0 = 49
M1 = 25
EP0 = 16 * K * M0
EP1 = 16 * K * M1
EP = EP0 + EP1
NC0 = EP0 // K
MD = EP // 16 // K

_MESH = dict(core_axis_name="c", subcore_axis_name="s",
             num_cores=2, num_subcores=16)
_CPARAMS = pltpu.CompilerParams(
    needs_layout_passes=False, use_tc_tiling_on_sc=False)


def _sc_scales(pidx, wts):
    def body(pidx_ref, wts_ref, sc2_ref,
             pbA, pbB, wbA, wbB, rudA, rudB, ridA, ridB, scbA, scbB, dtmp,
             sLA, sLB, sGA, sGB, sSA, sSB, smg, ud_s, id_s):
        core = lax.axis_index("c")
        sub = lax.axis_index("s")
        tb = sub * RPT
        z16 = jnp.zeros((16,), jnp.float32)

        @pl.loop(0, RPT // 16)
        def _(g):
            dtmp[pl.ds(g * 16, 16)] = z16

        pltpu.sync_copy(dtmp, ud_s.at[pl.ds(tb, RPT)])
        pltpu.sync_copy(dtmp, id_s.at[pl.ds(tb, RPT)])
        plsc.subcore_barrier()

        BS = ((pbA, wbA, rudA, ridA, scbA, sLA, sGA, sSA),
              (pbB, wbB, rudB, ridB, scbB, sLB, sGB, sSB))

        def issue_loads(c, bs):
            pb, wb, rud, rid, scb, sL, sG, sS = bs
            ch = sub * MD + c
            pltpu.async_copy(pidx_ref.at[ch], pb, sL)
            pltpu.async_copy(wts_ref.at[pl.ds(ch * K, K)], wb, sL)

        def wait_loads(bs):
            pb, wb, rud, rid, scb, sL, sG, sS = bs
            pltpu.make_async_copy(pidx_ref.at[0], pb, sL).wait()
            pltpu.make_async_copy(wts_ref.at[pl.ds(0, K)], wb, sL).wait()

        def deg_half(c, bs, nxt, first, last):
            pb, wb, rud, rid, scb, sL, sG, sS = bs
            if not first:
                pn, wn, _, _, _, _, _, sSn = nxt
                pltpu.make_async_copy(wn, ud_s.at[pl.ds(0, K)], sSn).wait()
                pltpu.make_async_copy(wn, id_s.at[pl.ds(0, K)], sSn).wait()
            wait_loads(bs)
            for j in range(KJ):
                wj = wb.at[pl.ds(j * 128, 128)]
                pltpu.async_copy(wj, ud_s.at[pb.at[0, j]], sS, add=True)
                pltpu.async_copy(wj, id_s.at[pb.at[1, j]], sS, add=True)
            if not last:
                issue_loads(c + 1, nxt)

        issue_loads(0, BS[0])
        deg_half(0, BS[0], BS[1], first=True, last=False)

        @pl.loop(0, MD // 2 - 1)
        def _(jj):
            c = 1 + 2 * jj
            deg_half(c, BS[1], BS[0], first=False, last=False)
            deg_half(c + 1, BS[0], BS[1], first=False, last=False)

        deg_half(MD - 1, BS[1], BS[0], first=False, last=True)
        pltpu.make_async_copy(wbB, ud_s.at[pl.ds(0, K)], sSB).wait()
        pltpu.make_async_copy(wbB, id_s.at[pl.ds(0, K)], sSB).wait()

        plsc.subcore_barrier()

        def rsqrt16(v):
            i = lax.bitcast_convert_type(v, jnp.int32)
            i = jnp.int32(0x5F3759DF) - lax.shift_right_logical(i, 1)
            y = lax.bitcast_convert_type(i, jnp.float32)
            for _ in range(3):
                y = y * (1.5 - 0.5 * v * y * y)
            return y

        for dref in (ud_s, id_s):
            pltpu.sync_copy(dref.at[pl.ds(tb, RPT)], dtmp)

            @pl.loop(0, RPT // 16)
            def _(g):
                v = jnp.maximum(dtmp[pl.ds(g * 16, 16)], 1.0)
                dtmp[pl.ds(g * 16, 16)] = rsqrt16(v)

            pltpu.sync_copy(dtmp, dref.at[pl.ds(tb, RPT)])
        plsc.subcore_barrier()

        def scl_half(c, bs, nxt, first, last):
            pb, wb, rud, rid, scb, sL, sG, sS = bs
            ch = sub * MD + c
            if not first:
                pn, wn, _, _, scn, _, _, sSn = nxt
                pltpu.make_async_copy(
                    scn, sc2_ref.at[core, pl.ds(0, K)], sSn).wait()
            wait_loads(bs)
            for j in range(KJ):
                pltpu.async_copy(
                    ud_s.at[pb.at[0, j]], rud.at[pl.ds(j * 128, 128)], sG)
                pltpu.async_copy(
                    id_s.at[pb.at[1, j]], rid.at[pl.ds(j * 128, 128)], sG)
            if not last:
                issue_loads(c + 1, nxt)
            pltpu.make_async_copy(ud_s.at[pl.ds(0, K)], rud, sG).wait()
            pltpu.make_async_copy(id_s.at[pl.ds(0, K)], rid, sG).wait()

            @pl.loop(0, K // 16, unroll=2)
            def _(g):
                sl = pl.ds(g * 16, 16)
                scb[sl] = wb[sl] * rud[sl] * rid[sl]

            pltpu.async_copy(scb, sc2_ref.at[core, pl.ds(ch * K, K)], sS)

        issue_loads(0, BS[0])
        scl_half(0, BS[0], BS[1], first=True, last=False)

        @pl.loop(0, MD // 2 - 1)
        def _(jj):
            c = 1 + 2 * jj
            scl_half(c, BS[1], BS[0], first=False, last=False)
            scl_half(c + 1, BS[0], BS[1], first=False, last=False)

        scl_half(MD - 1, BS[1], BS[0], first=False, last=True)
        pltpu.make_async_copy(scbB, sc2_ref.at[core, pl.ds(0, K)], sSB).wait()

    fn = pl.kernel(
        body,
        out_type=jax.ShapeDtypeStruct((2, EP), jnp.float32),
        mesh=plsc.VectorSubcoreMesh(**_MESH),
        compiler_params=_CPARAMS,
        scratch_types=[
            pltpu.VMEM((2, KJ, 128), jnp.int32),
            pltpu.VMEM((2, KJ, 128), jnp.int32),
            pltpu.VMEM((K,), jnp.float32),
            pltpu.VMEM((K,), jnp.float32),
            pltpu.VMEM((K,), jnp.float32),
            pltpu.VMEM((K,), jnp.float32),
            pltpu.VMEM((K,), jnp.float32),
            pltpu.VMEM((K,), jnp.float32),
            pltpu.VMEM((K,), jnp.float32),
            pltpu.VMEM((K,), jnp.float32),
            pltpu.VMEM((RPT,), jnp.float32),
            pltpu.SemaphoreType.DMA,
            pltpu.SemaphoreType.DMA,
            pltpu.SemaphoreType.DMA,
            pltpu.SemaphoreType.DMA,
            pltpu.SemaphoreType.DMA,
            pltpu.SemaphoreType.DMA,
            pltpu.SemaphoreType.DMA,
            pltpu.VMEM_SHARED((NVP,), jnp.float32),
            pltpu.VMEM_SHARED((NVP,), jnp.float32),
        ],
    )
    return fn(pidx, wts)


def _sc_messages(pidx, sc2, rflat, ucat, icat):
    def body(pidx_ref, sc2_ref, rflat_ref, ucat_ref, icat_ref, msg_ref,
             pbA, pbB, rwA, rwB, scA, scB, rbuf,
             sIA, sIB, sRA, sRB, sOA, sOB, smg, acc):
        core = lax.axis_index("c")
        sub = lax.axis_index("s")
        tb = sub * RPT
        z16 = jnp.zeros((16,), jnp.float32)

        pltpu.sync_copy(rflat_ref, rbuf)

        def edge_loop(n, chbase, tbl, dsel, gsel, out_side, rv, poff):
            BS = ((pbA, scA, rwA, sIA, sRA, sOA),
                  (pbB, scB, rwB, sIB, sRB, sOB))

            def issue_idx(c, bs):
                pb, sc, rw, sI, sR, sO = bs
                ch = chbase + c
                pltpu.async_copy(pidx_ref.at[ch], pb, sI)
                pltpu.async_copy(
                    sc2_ref.at[out_side, pl.ds(ch * K, K)], sc, sI)

            def wait_idx(bs):
                pb, sc, rw, sI, sR, sO = bs
                pltpu.make_async_copy(pidx_ref.at[chbase], pb, sI).wait()
                pltpu.make_async_copy(
                    sc2_ref.at[out_side, pl.ds(chbase * K, K)], sc, sI).wait()

            def issue_gathers(bs):
                pb, sc, rw, sI, sR, sO = bs
                for j in range(KJ):
                    pltpu.async_copy(tbl.at[pb.at[gsel, j]],
                                     rw.at[pl.ds(j * 128, 128)], sR)

            def wait_gathers(bs):
                pb, sc, rw, sI, sR, sO = bs
                pltpu.make_async_copy(tbl.at[pl.ds(0, K)], rw, sR).wait()

            def issue_scatter(bs):
                pb, sc, rw, sI, sR, sO = bs
                for j in range(KJ):
                    pltpu.async_copy(rw.at[pl.ds(j * 128, 128)],
                                     acc.at[pb.at[dsel, j]], sO, add=True)

            def wait_scatter(bs):
                pb, sc, rw, sI, sR, sO = bs
                pltpu.make_async_copy(rw, acc.at[pl.ds(0, K)], sO).wait()

            def compute(bs):
                pb, sc, rw, sI, sR, sO = bs

                @pl.loop(0, K // 16, unroll=2)
                def _(g):
                    sv16 = sc[pl.ds(g * 16, 16)]
                    for l in range(16):
                        r = g * 16 + l
                        sv = lax.broadcast(sv16[l], (16,))
                        rw[r, pl.ds(0, 16)] = rw[r, pl.ds(0, 16)] * (sv * rv)

            def adjust_idx(bs):
                pb = bs[0]
                for j in range(KJ):
                    @pl.loop(0, 128 // 16)
                    def _(g):
                        sl = pl.ds(g * 16, 16)
                        v = lax.shift_left(pb[gsel, j, sl], 3)
                        pb[gsel, j, sl] = v + poff

            def half(c, bs, bs_next, first, last):
                if not first:
                    wait_scatter(bs_next)
                if not last:
                    issue_idx(c + 1, bs_next)
                wait_gathers(bs)
                compute(bs)
                issue_scatter(bs)
                if not last:
                    wait_idx(bs_next)
                    adjust_idx(bs_next)
                    issue_gathers(bs_next)

            issue_idx(0, BS[0])
            wait_idx(BS[0])
            adjust_idx(BS[0])
            issue_gathers(BS[0])
            half(0, BS[0], BS[1], first=True, last=False)

            @pl.loop(0, (n - 2) // 2)
            def _(jj):
                c = 1 + 2 * jj
                half(c, BS[1], BS[0], first=False, last=False)
                half(c + 1, BS[0], BS[1], first=False, last=False)

            if n % 2 == 0:
                half(n - 1, BS[1], BS[0], first=False, last=True)
                wait_scatter(BS[1])
            else:
                half(n - 2, BS[1], BS[0], first=False, last=False)
                half(n - 1, BS[0], BS[1], first=False, last=True)
                wait_scatter(BS[0])

        def side(tbl, dsel, gsel, out_side):
            nfull = RPT // K
            rem = RPT - nfull * K

            @pl.loop(0, NCH)
            def _(p):
                poff = lax.broadcast(p, (16,)).astype(jnp.int32)

                @pl.loop(0, K)
                def _(r):
                    rwA[r, pl.ds(0, 16)] = z16

                ds_ = []
                for q in range(nfull):
                    ds_.append(pltpu.async_copy(
                        rwA, acc.at[pl.ds(tb + q * K, K)], smg))
                if rem:
                    ds_.append(pltpu.async_copy(
                        rwA.at[pl.ds(0, rem)],
                        acc.at[pl.ds(tb + nfull * K, rem)], smg))
                for d in ds_:
                    d.wait()
                plsc.subcore_barrier()

                rv_t0 = rbuf[pl.ds(p * CH, 16)]
                rv_t1 = rbuf[pl.ds(EMB + p * CH, 16)]
                edge_loop(M0, sub * M0, tbl, dsel, gsel, out_side, rv_t0,
                          poff)
                edge_loop(M1, NC0 + sub * M1, tbl, dsel, gsel, out_side,
                          rv_t1, poff)
                plsc.subcore_barrier()
                pltpu.sync_copy(acc.at[pl.ds(tb, RPT)],
                                msg_ref.at[out_side, p, pl.ds(tb, RPT)])

        @pl.when(core == 0)
        def _():
            side(icat_ref, 0, 1, 0)

        @pl.when(core == 1)
        def _():
            side(ucat_ref, 1, 0, 1)

    fn = pl.kernel(
        body,
        out_type=jax.ShapeDtypeStruct((2, NCH, NVP, CH), jnp.float32),
        mesh=plsc.VectorSubcoreMesh(**_MESH),
        compiler_params=_CPARAMS,
        scratch_types=[
            pltpu.VMEM((2, KJ, 128), jnp.int32),
            pltpu.VMEM((2, KJ, 128), jnp.int32),
            pltpu.VMEM((K, CH), jnp.float32),
            pltpu.VMEM((K, CH), jnp.float32),
            pltpu.VMEM((K,), jnp.float32),
            pltpu.VMEM((K,), jnp.float32),
            pltpu.VMEM((2 * EMB,), jnp.float32),
            pltpu.SemaphoreType.DMA,
            pltpu.SemaphoreType.DMA,
            pltpu.SemaphoreType.DMA,
            pltpu.SemaphoreType.DMA,
            pltpu.SemaphoreType.DMA,
            pltpu.SemaphoreType.DMA,
            pltpu.SemaphoreType.DMA,
            pltpu.VMEM_SHARED((NVP, CH), jnp.float32),
        ],
    )
    return fn(pidx, sc2, rflat, ucat, icat)


def _tc_matmul_body(x_ref, w_ref, ou_ref, oi_ref):
    for s, o_ref in ((0, ou_ref), (1, oi_ref)):
        x = x_ref[s]
        m = jnp.concatenate([x[p] for p in range(NCH)], axis=1)
        y = lax.dot_general(m, w_ref[s], (((1,), (1,)), ((), ())),
                            preferred_element_type=jnp.float32)
        o_ref[...] = jax.nn.sigmoid(y)


def _tc_rel_body(r_ref, w_ref, o_ref):
    o_ref[...] = lax.dot_general(r_ref[...], w_ref[...],
                                 (((1,), (1,)), ((), ())),
                                 preferred_element_type=jnp.float32)


def kernel(u_emb, i_emb, r_emb, edge_t0_index, edge_t0_weights,
           edge_t1_index, edge_t1_weights, W_u, W_i, W_rel):
    def pad_edges(idx, w, ept):
        e = w.shape[0]
        spread = (jnp.arange(ept - e, dtype=jnp.int32) * 97) % NV
        pu = jnp.concatenate([idx[0], spread])
        pi = jnp.concatenate([idx[1], spread])
        pw = jnp.concatenate([w, jnp.zeros((ept - e,), jnp.float32)])
        return pu, pi, pw

    pu0, pi0, pw0 = pad_edges(edge_t0_index, edge_t0_weights, EP0)
    pu1, pi1, pw1 = pad_edges(edge_t1_index, edge_t1_weights, EP1)
    uidx = jnp.concatenate([pu0, pu1])
    iidx = jnp.concatenate([pi0, pi1])
    pidx = (jnp.stack([uidx, iidx])
            .reshape(2, EP // K, KJ * 128)
            .transpose(1, 0, 2)
            .reshape(EP // K, 2, KJ, 128))
    wts = jnp.concatenate([pw0, pw1])
    rflat = r_emb.reshape(2 * EMB)
    ucat = u_emb.reshape(NCH * NV, CH)
    icat = i_emb.reshape(NCH * NV, CH)

    sc2 = _sc_scales(pidx, wts)
    msg = _sc_messages(pidx, sc2, rflat, ucat, icat)

    B = 1024
    nblk = NVP // B
    w_st = jnp.stack([W_u, W_i])
    new_u, new_i = pl.pallas_call(
        _tc_matmul_body,
        grid=(nblk,),
        in_specs=[
            pl.BlockSpec((2, NCH, B, CH), lambda i: (0, 0, i, 0)),
            pl.BlockSpec((2, EMB, EMB), lambda i: (0, 0, 0)),
        ],
        out_specs=[
            pl.BlockSpec((B, EMB), lambda i: (i, 0)),
            pl.BlockSpec((B, EMB), lambda i: (i, 0)),
        ],
        out_shape=[
            jax.ShapeDtypeStruct((NV, EMB), jnp.float32),
            jax.ShapeDtypeStruct((NV, EMB), jnp.float32),
        ],
    )(msg, w_st)

    new_r = pl.pallas_call(
        _tc_rel_body,
        out_shape=jax.ShapeDtypeStruct((2, EMB), jnp.float32),
    )(r_emb, W_rel)

    return new_u, new_i, new_r
